# baseline (speedup 1.0000x reference)
import functools

import jax
import jax.numpy as jnp
from jax import lax
from jax.experimental import pallas as pl
from jax.experimental.pallas import tpu as pltpu
from jax.experimental.pallas import tpu_sc as plsc

H = 8
VOCAB = 100000
DIM = 64
B = 1024
T = 50
NTOK = B * T
NC = 2
NS = 16
NW = NC * NS
TPW = NTOK // NW
CK = 80
HCK = TPW // 2
NCH = HCK // CK
GP16 = HCK // 16

_mesh = plsc.VectorSubcoreMesh(core_axis_name="c", subcore_axis_name="s")
_sc_params = pltpu.CompilerParams(use_tc_tiling_on_sc=False)


@functools.partial(
    pl.kernel,
    mesh=_mesh,
    compiler_params=_sc_params,
    out_type=jax.ShapeDtypeStruct((NTOK,), jnp.int32),
    scratch_types=[
        pltpu.VMEM((TPW,), jnp.int32),
        pltpu.VMEM((TPW,), jnp.int32),
    ],
)
def _mask_kernel(codecs_hbm, mask_hbm, idx_v, macc_v):
    wid = lax.axis_index("s") * NC + lax.axis_index("c")
    tok0 = wid * TPW

    def init_mask(g, _):
        macc_v[pl.ds(g * 16, 16)] = jnp.full((16,), 1, jnp.int32)
        return 0

    lax.fori_loop(0, TPW // 16, init_mask, 0)

    def head(h, _):
        pltpu.sync_copy(codecs_hbm.at[pl.ds(h * NTOK + tok0, TPW)], idx_v)

        def scan(g, _):
            off = g * 16
            eq = jnp.where(idx_v[pl.ds(off, 16)] == 0, 1, 0).astype(jnp.int32)
            macc_v[pl.ds(off, 16)] = macc_v[pl.ds(off, 16)] & eq
            return 0

        lax.fori_loop(0, TPW // 16, scan, 0)
        return 0

    lax.fori_loop(0, H, head, 0)
    pltpu.sync_copy(macc_v, mask_hbm.at[pl.ds(tok0, TPW)])


@functools.partial(
    pl.kernel,
    mesh=_mesh,
    compiler_params=_sc_params,
    out_type=jax.ShapeDtypeStruct((NTOK, DIM), jnp.float32),
    scratch_types=[
        pltpu.VMEM((TPW,), jnp.int32),
        pltpu.VMEM((HCK, DIM), jnp.float32),
        pltpu.VMEM((HCK, DIM), jnp.float32),
        pltpu.SemaphoreType.DMA,
        pltpu.SemaphoreType.DMA,
        pltpu.SemaphoreType.DMA,
    ],
)
def _gather_head(ids_hbm, w_hbm, emb_hbm, idx_v, rows0_v, rows1_v,
                 gsem, osem0, osem1):
    wid = lax.axis_index("s") * NC + lax.axis_index("c")
    tok0 = wid * TPW
    pltpu.sync_copy(ids_hbm.at[pl.ds(tok0, TPW)], idx_v)

    for p, (rows_v, osem) in enumerate(((rows0_v, osem0), (rows1_v, osem1))):
        h0 = p * HCK

        def scan(g, orv, h0=h0):
            v = idx_v[pl.ds(h0 + g * 16, 16)]
            return orv | jnp.where(v == 0, 1, 0).astype(jnp.int32)

        orv = lax.fori_loop(0, GP16, scan, jnp.zeros((16,), jnp.int32))
        any_zero = orv[0]
        for l in range(1, 16):
            any_zero = any_zero | orv[l]

        descs = []
        for j in range(NCH):
            descs.append(pltpu.async_copy(
                w_hbm.at[idx_v.at[pl.ds(h0 + j * CK, CK)]],
                rows_v.at[pl.ds(j * CK, CK)], gsem))
        for d in descs:
            d.wait()

        @pl.when(any_zero > 0)
        def _fix(rows_v=rows_v, h0=h0):
            zeros = jnp.zeros((16,), jnp.float32)

            def fix_group(g, _):
                v16 = idx_v[pl.ds(h0 + g * 16, 16)]
                for l in range(16):
                    @pl.when(v16[l] == 0)
                    def _z(l=l):
                        for q in range(DIM // 16):
                            rows_v[g * 16 + l, pl.ds(q * 16, 16)] = zeros
                return 0

            lax.fori_loop(0, GP16, fix_group, 0)

        pltpu.async_copy(rows_v, emb_hbm.at[pl.ds(tok0 + h0, HCK)], osem)

    for rows_v, osem in ((rows0_v, osem0), (rows1_v, osem1)):
        pltpu.make_async_copy(
            rows_v, emb_hbm.at[pl.ds(0, HCK)], osem).wait()


def kernel(codecs, W):
    codecs_t = jnp.transpose(codecs.reshape(NTOK, H)).reshape(H * NTOK)
    mask_i32 = _mask_kernel(codecs_t)
    embs = []
    for h in range(H):
        ids_h = lax.slice(codecs_t, (h * NTOK,), ((h + 1) * NTOK,))
        emb_h = _gather_head(ids_h, W[h])
        embs.append(emb_h.reshape(B, T, DIM))
    emb = jnp.stack(embs)
    mask = mask_i32.reshape(B, T).astype(bool)
    return (emb, mask)


# speedup vs baseline: 1.4000x; 1.4000x over previous
import functools

import jax
import jax.numpy as jnp
from jax import lax
from jax.experimental import pallas as pl
from jax.experimental.pallas import tpu as pltpu
from jax.experimental.pallas import tpu_sc as plsc

H = 8
VOCAB = 100000
DIM = 64
B = 1024
T = 50
NTOK = B * T
NC = 2
NS = 16
NW = NC * NS
TPW = NTOK // NW
CK = 80
HCK = TPW // 2
NCH = HCK // CK
GP16 = HCK // 16

_mesh = plsc.VectorSubcoreMesh(core_axis_name="c", subcore_axis_name="s")


@functools.partial(
    pl.kernel,
    mesh=_mesh,
    compiler_params=pltpu.CompilerParams(use_tc_tiling_on_sc=False),
    out_type=[
        jax.ShapeDtypeStruct((H * NTOK, DIM), jnp.float32),
        jax.ShapeDtypeStruct((NTOK,), jnp.int32),
    ],
    scratch_types=[
        pltpu.VMEM((TPW,), jnp.int32),
        pltpu.VMEM((TPW,), jnp.int32),
        pltpu.VMEM((HCK, DIM), jnp.float32),
        pltpu.VMEM((HCK, DIM), jnp.float32),
        pltpu.SemaphoreType.DMA,
        pltpu.SemaphoreType.DMA,
        pltpu.SemaphoreType.DMA,
        pltpu.SemaphoreType.DMA,
    ],
)
def _emb_kernel(codecs_hbm, w_hbm, emb_hbm, mask_hbm,
                idx_v, macc_v, rows0_v, rows1_v, gsem0, gsem1, osem0, osem1):
    wid = lax.axis_index("s") * NC + lax.axis_index("c")
    tok0 = wid * TPW
    rows_bufs = (rows0_v, rows1_v)
    gsems = (gsem0, gsem1)
    osems = (osem0, osem1)

    def init_mask(g, _):
        macc_v[pl.ds(g * 16, 16)] = jnp.full((16,), 1, jnp.int32)
        return 0

    lax.fori_loop(0, TPW // 16, init_mask, 0)

    def head(h, _):
        base = h * VOCAB
        pltpu.sync_copy(codecs_hbm.at[pl.ds(h * NTOK + tok0, TPW)], idx_v)

        any_zero = [None, None]
        descs = [None, None]
        for p in (0, 1):
            h0 = p * HCK

            def bias(g, orv, h0=h0):
                off = h0 + g * 16
                v = idx_v[pl.ds(off, 16)]
                eq = jnp.where(v == 0, 1, 0).astype(jnp.int32)
                idx_v[pl.ds(off, 16)] = v + base
                macc_v[pl.ds(off, 16)] = macc_v[pl.ds(off, 16)] & eq
                return orv | eq

            orv = lax.fori_loop(0, GP16, bias, jnp.zeros((16,), jnp.int32))
            az = orv[0]
            for l in range(1, 16):
                az = az | orv[l]
            any_zero[p] = az

            @pl.when(h > 0)
            def _drain_prev(p=p):
                pltpu.make_async_copy(
                    rows_bufs[p], emb_hbm.at[pl.ds(0, HCK)], osems[p]).wait()

            descs[p] = [
                pltpu.async_copy(
                    w_hbm.at[idx_v.at[pl.ds(h0 + j * CK, CK)]],
                    rows_bufs[p].at[pl.ds(j * CK, CK)], gsems[p])
                for j in range(NCH)
            ]

        for p in (0, 1):
            rows_v = rows_bufs[p]
            h0 = p * HCK
            for d in descs[p]:
                d.wait()

            @pl.when(any_zero[p] > 0)
            def _fix(rows_v=rows_v, h0=h0, base=base):
                zeros = jnp.zeros((16,), jnp.float32)

                def fix_group(g, _):
                    v16 = idx_v[pl.ds(h0 + g * 16, 16)]
                    for l in range(16):
                        @pl.when(v16[l] == base)
                        def _z(l=l):
                            for q in range(DIM // 16):
                                rows_v[g * 16 + l, pl.ds(q * 16, 16)] = zeros
                    return 0

                lax.fori_loop(0, GP16, fix_group, 0)

            pltpu.async_copy(
                rows_v,
                emb_hbm.at[pl.ds(h * NTOK + tok0 + h0, HCK)], osems[p])
        return 0

    lax.fori_loop(0, H, head, 0)

    for p in (0, 1):
        pltpu.make_async_copy(
            rows_bufs[p], emb_hbm.at[pl.ds(0, HCK)], osems[p]).wait()

    pltpu.sync_copy(macc_v, mask_hbm.at[pl.ds(tok0, TPW)])


def kernel(codecs, W):
    codecs_t = jnp.transpose(codecs.reshape(NTOK, H)).reshape(H * NTOK)
    w_flat = W.reshape(H * VOCAB, DIM)
    emb, mask_i32 = _emb_kernel(codecs_t, w_flat)
    emb = emb.reshape(H, B, T, DIM)
    mask = mask_i32.reshape(B, T).astype(bool)
    return (emb, mask)
